# baseline (speedup 1.0000x reference)
import functools

import jax
import jax.numpy as jnp
from jax import lax
from jax.experimental import pallas as pl
from jax.experimental.pallas import tpu as pltpu
from jax.experimental.pallas import tpu_sc as plsc

_B = 128
_L = 8192
_QS = 64
_KS = 64
_BN = 64
_VD = 64
_TOPK = 20
_KP = 24
_BB = 4
_TB = 64
_NW = 32
_BPW = _B // _NW
_GPB = _KP * _VD
_GCH = 128


def _energy_body(qt_ref, kt_ref, Wt_ref, Ut_ref, bc_ref, wc_ref,
                 eo_ref, wt_ref):
    wht = jnp.dot(Wt_ref[...].astype(jnp.bfloat16),
                  qt_ref[...].astype(jnp.bfloat16),
                  preferred_element_type=jnp.float32)
    ut = Ut_ref[...].astype(jnp.bfloat16)
    bc = bc_ref[...]
    wc = wc_ref[...].astype(jnp.bfloat16).astype(jnp.float32)
    col = lax.broadcasted_iota(jnp.int32, (_BN, _B), 1)
    rows = []
    for i in range(_BB):
        gi = pl.program_id(0) * _BB + i
        whi = jnp.sum(jnp.where(col == gi, wht, 0.0), axis=1, keepdims=True)
        a = kt_ref[i].astype(jnp.bfloat16)
        uvt = jnp.dot(ut, a, preferred_element_type=jnp.float32)
        th = jnp.tanh(whi + uvt + bc)
        tb = th.astype(jnp.bfloat16).astype(jnp.float32)
        rows.append(jnp.sum(tb * wc, axis=0, keepdims=True))
    e = jnp.concatenate(rows, axis=0)
    m = jnp.max(e, axis=1, keepdims=True)
    ex = jnp.exp(e - m)
    s = jnp.sum(ex, axis=1, keepdims=True)
    eo_ref[...] = e.reshape(_BB * 64, 128)
    wt_ref[...] = (ex / s).reshape(_BB * 64, 128)


def _idx_body(eo_ref, gr_ref, wr_ref, or_ref):
    e = eo_ref[...].reshape(_TB, _L)
    m = jnp.max(e, axis=1, keepdims=True)
    ex = jnp.exp(e - m)
    s = jnp.sum(ex, axis=1, keepdims=True)

    iota = lax.broadcasted_iota(jnp.int32, (_TB, _L), 1)
    k_iota = lax.broadcasted_iota(jnp.int32, (_TB, _KP), 1)
    ecur = e
    topv = jnp.full((_TB, _KP), -jnp.inf, jnp.float32)
    topi = jnp.zeros((_TB, _KP), jnp.int32)
    for k in range(_TOPK):
        mv = jnp.max(ecur, axis=1, keepdims=True)
        cand = jnp.where(ecur == mv, iota, jnp.int32(_L))
        mi = jnp.min(cand, axis=1, keepdims=True)
        topv = jnp.where(k_iota == k, mv, topv)
        topi = jnp.where(k_iota == k, mi, topi)
        ecur = jnp.where(iota == mi, -jnp.inf, ecur)

    topw = jnp.where(topv == -jnp.inf, 0.0, jnp.exp(topv - m) / s)
    wr_ref[...] = jnp.broadcast_to(topw[:, :, None], (_TB, _KP, 16))

    b3 = (pl.program_id(0) * _TB
          + lax.broadcasted_iota(jnp.int32, (_TB, _KP, _VD), 0))
    v3 = lax.broadcasted_iota(jnp.int32, (_TB, _KP, _VD), 2)
    l3 = topi[:, :, None]
    gr_ref[...] = ((((b3 * 8 + (v3 >> 3)) * 64 + (l3 >> 7)) * 8
                    + (v3 & 7)) * 8 + ((l3 >> 4) & 7))
    or_ref[...] = jnp.broadcast_to((topi & 15)[:, :, None], (_TB, _KP, 16))


def _tc_stage(query, keys, W, U, b, w, interpret=False):
    keys_t = jnp.transpose(keys, (0, 2, 1))
    query_t = query.T
    e2d, w2d = pl.pallas_call(
        _energy_body,
        grid=(_B // _BB,),
        in_specs=[
            pl.BlockSpec((_QS, _B), lambda i: (0, 0)),
            pl.BlockSpec((_BB, _KS, _L), lambda i: (i, 0, 0)),
            pl.BlockSpec((_BN, _QS), lambda i: (0, 0)),
            pl.BlockSpec((_BN, _KS), lambda i: (0, 0)),
            pl.BlockSpec((_BN, 1), lambda i: (0, 0)),
            pl.BlockSpec((_BN, 1), lambda i: (0, 0)),
        ],
        out_specs=[
            pl.BlockSpec((_BB * 64, 128), lambda i: (i, 0)),
            pl.BlockSpec((_BB * 64, 128), lambda i: (i, 0)),
        ],
        out_shape=[
            jax.ShapeDtypeStruct((_B * 64, 128), jnp.float32),
            jax.ShapeDtypeStruct((_B * 64, 128), jnp.float32),
        ],
        interpret=interpret,
    )(query_t, keys_t, W.T, U.T, b.reshape(_BN, 1), w.reshape(_BN, 1))

    grows, wrep, orep = pl.pallas_call(
        _idx_body,
        grid=(_B // _TB,),
        in_specs=[pl.BlockSpec((_TB * 64, 128), lambda i: (i, 0))],
        out_specs=[
            pl.BlockSpec((_TB, _KP, _VD), lambda i: (i, 0, 0)),
            pl.BlockSpec((_TB, _KP, 16), lambda i: (i, 0, 0)),
            pl.BlockSpec((_TB, _KP, 16), lambda i: (i, 0, 0)),
        ],
        out_shape=[
            jax.ShapeDtypeStruct((_B, _KP, _VD), jnp.int32),
            jax.ShapeDtypeStruct((_B, _KP, 16), jnp.float32),
            jax.ShapeDtypeStruct((_B, _KP, 16), jnp.int32),
        ],
        interpret=interpret,
    )(e2d)
    return e2d, w2d, grows, wrep, orep


def _sc_gather(values, grows_flat, wrep_flat, orep_flat):
    vview = (values.transpose(0, 2, 1)
             .reshape(_B, 8, 8, 64, 128)
             .transpose(0, 1, 3, 2, 4)
             .reshape(_B * 8 * 64 * 8 * 8, 16))
    mesh = plsc.VectorSubcoreMesh(core_axis_name="c", subcore_axis_name="s")

    @functools.partial(
        pl.kernel,
        mesh=mesh,
        out_type=jax.ShapeDtypeStruct((_B, _VD), jnp.float32),
        compiler_params=pltpu.CompilerParams(use_tc_tiling_on_sc=False,
                                             needs_layout_passes=False),
        scratch_types=[
            pltpu.VMEM((_BPW * _GPB,), jnp.int32),
            pltpu.VMEM((_BPW * _GPB, 16), jnp.float32),
            pltpu.VMEM((_BPW * _KP, 16), jnp.float32),
            pltpu.VMEM((_BPW * _KP, 16), jnp.int32),
            pltpu.VMEM((_BPW, _VD), jnp.float32),
            pltpu.SemaphoreType.DMA,
        ],
    )
    def run(vview_hbm, grows_hbm, wrep_hbm, orep_hbm, out_hbm,
            idx_v, rows_v, w_v, off_v, acc_v, sem):
        wid = lax.axis_index("s") * 2 + lax.axis_index("c")
        pltpu.sync_copy(grows_hbm.at[pl.ds(wid * _BPW * _GPB, _BPW * _GPB)],
                        idx_v)
        pltpu.sync_copy(wrep_hbm.at[pl.ds(wid * _BPW * _KP, _BPW * _KP)], w_v)
        pltpu.sync_copy(orep_hbm.at[pl.ds(wid * _BPW * _KP, _BPW * _KP)],
                        off_v)
        cps = []
        for c in range(_BPW * _GPB // _GCH):
            cps.append(pltpu.async_copy(
                vview_hbm.at[idx_v.at[pl.ds(c * _GCH, _GCH)]],
                rows_v.at[pl.ds(c * _GCH, _GCH)], sem))
        for cp in cps:
            cp.wait()
        for j in range(_BPW):
            for vc in range(_VD // 16):
                acc = jnp.zeros((16,), jnp.float32)
                rowv = lax.iota(jnp.int32, 16) + (j * _GPB + vc * 16)
                for k in range(_KP):
                    vals = plsc.load_gather(
                        rows_v, [rowv + k * _VD, off_v[j * _KP + k]])
                    acc = acc + vals * w_v[j * _KP + k]
                acc_v[j, pl.ds(vc * 16, 16)] = acc
        pltpu.sync_copy(acc_v, out_hbm.at[pl.ds(wid * _BPW, _BPW)])

    return run(vview, grows_flat, wrep_flat, orep_flat)


def kernel(query, keys, values, W, U, b, w):
    e2d, w2d, grows, wrep, orep = _tc_stage(query, keys, W, U, b, w)
    attn = _sc_gather(values,
                      grows.reshape(_B * _GPB),
                      wrep.reshape(_B * _KP, 16),
                      orep.reshape(_B * _KP, 16))
    return (attn, w2d.reshape(_B, _L, 1), e2d.reshape(_B, _L, 1))


# speedup vs baseline: 1.0599x; 1.0599x over previous
import functools

import jax
import jax.numpy as jnp
from jax import lax
from jax.experimental import pallas as pl
from jax.experimental.pallas import tpu as pltpu
from jax.experimental.pallas import tpu_sc as plsc

_B = 128
_L = 8192
_QS = 64
_KS = 64
_BN = 64
_VD = 64
_TOPK = 20
_KP = 24
_BB = 8
_TB = 64
_NW = 32
_BPW = _B // _NW
_GPB = _KP * _VD
_GCH = 128


def _energy_body(qt_ref, kt_ref, Wt_ref, Ut_ref, bc_ref, wc_ref,
                 eo_ref, wt_ref):
    wht = jnp.dot(Wt_ref[...].astype(jnp.bfloat16),
                  qt_ref[...].astype(jnp.bfloat16),
                  preferred_element_type=jnp.float32)
    ut = Ut_ref[...].astype(jnp.bfloat16)
    bc = bc_ref[...]
    wc = wc_ref[...].astype(jnp.bfloat16).astype(jnp.float32)
    col = lax.broadcasted_iota(jnp.int32, (_BN, _B), 1)
    rows = []
    for i in range(_BB):
        gi = pl.program_id(0) * _BB + i
        whi = jnp.sum(jnp.where(col == gi, wht, 0.0), axis=1, keepdims=True)
        a = kt_ref[i].astype(jnp.bfloat16)
        uvt = jnp.dot(ut, a, preferred_element_type=jnp.float32)
        th = jnp.tanh(whi + uvt + bc)
        tb = th.astype(jnp.bfloat16).astype(jnp.float32)
        rows.append(jnp.sum(tb * wc, axis=0, keepdims=True))
    e = jnp.concatenate(rows, axis=0)
    m = jnp.max(e, axis=1, keepdims=True)
    ex = jnp.exp(e - m)
    s = jnp.sum(ex, axis=1, keepdims=True)
    eo_ref[...] = e.reshape(_BB * 64, 128)
    wt_ref[...] = (ex / s).reshape(_BB * 64, 128)


def _idx_body(eo_ref, gr_ref, wr_ref, or_ref):
    e = eo_ref[...].reshape(_TB, _L)
    m = jnp.max(e, axis=1, keepdims=True)
    ex = jnp.exp(e - m)
    s = jnp.sum(ex, axis=1, keepdims=True)

    iota = lax.broadcasted_iota(jnp.int32, (_TB, _L), 1)
    k_iota = lax.broadcasted_iota(jnp.int32, (_TB, _KP), 1)
    ecur = e
    topv = jnp.full((_TB, _KP), -jnp.inf, jnp.float32)
    topi = jnp.zeros((_TB, _KP), jnp.int32)
    for k in range(_TOPK):
        mv = jnp.max(ecur, axis=1, keepdims=True)
        cand = jnp.where(ecur == mv, iota, jnp.int32(_L))
        mi = jnp.min(cand, axis=1, keepdims=True)
        topv = jnp.where(k_iota == k, mv, topv)
        topi = jnp.where(k_iota == k, mi, topi)
        ecur = jnp.where(iota == mi, -jnp.inf, ecur)

    topw = jnp.where(topv == -jnp.inf, 0.0, jnp.exp(topv - m) / s)
    wr_ref[...] = jnp.broadcast_to(topw[:, :, None], (_TB, _KP, 16))

    b3 = (pl.program_id(0) * _TB
          + lax.broadcasted_iota(jnp.int32, (_TB, _KP, _VD), 0))
    v3 = lax.broadcasted_iota(jnp.int32, (_TB, _KP, _VD), 2)
    l3 = topi[:, :, None]
    gr_ref[...] = ((((b3 * 8 + (v3 >> 3)) * 64 + (l3 >> 7)) * 8
                    + (v3 & 7)) * 8 + ((l3 >> 4) & 7))
    or_ref[...] = jnp.broadcast_to((topi & 15)[:, :, None], (_TB, _KP, 16))


def _tc_stage(query, keys, W, U, b, w, interpret=False):
    keys_t = jnp.transpose(keys, (0, 2, 1))
    query_t = query.T
    e2d, w2d = pl.pallas_call(
        _energy_body,
        grid=(_B // _BB,),
        in_specs=[
            pl.BlockSpec((_QS, _B), lambda i: (0, 0)),
            pl.BlockSpec((_BB, _KS, _L), lambda i: (i, 0, 0)),
            pl.BlockSpec((_BN, _QS), lambda i: (0, 0)),
            pl.BlockSpec((_BN, _KS), lambda i: (0, 0)),
            pl.BlockSpec((_BN, 1), lambda i: (0, 0)),
            pl.BlockSpec((_BN, 1), lambda i: (0, 0)),
        ],
        out_specs=[
            pl.BlockSpec((_BB * 64, 128), lambda i: (i, 0)),
            pl.BlockSpec((_BB * 64, 128), lambda i: (i, 0)),
        ],
        out_shape=[
            jax.ShapeDtypeStruct((_B * 64, 128), jnp.float32),
            jax.ShapeDtypeStruct((_B * 64, 128), jnp.float32),
        ],
        compiler_params=pltpu.CompilerParams(
            vmem_limit_bytes=56 * 1024 * 1024),
        interpret=interpret,
    )(query_t, keys_t, W.T, U.T, b.reshape(_BN, 1), w.reshape(_BN, 1))

    grows, wrep, orep = pl.pallas_call(
        _idx_body,
        grid=(_B // _TB,),
        in_specs=[pl.BlockSpec((_TB * 64, 128), lambda i: (i, 0))],
        out_specs=[
            pl.BlockSpec((_TB, _KP, _VD), lambda i: (i, 0, 0)),
            pl.BlockSpec((_TB, _KP, 16), lambda i: (i, 0, 0)),
            pl.BlockSpec((_TB, _KP, 16), lambda i: (i, 0, 0)),
        ],
        out_shape=[
            jax.ShapeDtypeStruct((_B, _KP, _VD), jnp.int32),
            jax.ShapeDtypeStruct((_B, _KP, 16), jnp.float32),
            jax.ShapeDtypeStruct((_B, _KP, 16), jnp.int32),
        ],
        interpret=interpret,
    )(e2d)
    return e2d, w2d, grows, wrep, orep


def _sc_gather(values, grows_flat, wrep_flat, orep_flat):
    vview = (values.transpose(0, 2, 1)
             .reshape(_B, 8, 8, 64, 128)
             .transpose(0, 1, 3, 2, 4)
             .reshape(_B * 8 * 64 * 8 * 8, 16))
    mesh = plsc.VectorSubcoreMesh(core_axis_name="c", subcore_axis_name="s")

    @functools.partial(
        pl.kernel,
        mesh=mesh,
        out_type=jax.ShapeDtypeStruct((_B, _VD), jnp.float32),
        compiler_params=pltpu.CompilerParams(use_tc_tiling_on_sc=False,
                                             needs_layout_passes=False),
        scratch_types=[
            pltpu.VMEM((_BPW * _GPB,), jnp.int32),
            pltpu.VMEM((_BPW * _GPB, 16), jnp.float32),
            pltpu.VMEM((_BPW * _KP, 16), jnp.float32),
            pltpu.VMEM((_BPW * _KP, 16), jnp.int32),
            pltpu.VMEM((_BPW, _VD), jnp.float32),
            pltpu.SemaphoreType.DMA,
        ],
    )
    def run(vview_hbm, grows_hbm, wrep_hbm, orep_hbm, out_hbm,
            idx_v, rows_v, w_v, off_v, acc_v, sem):
        wid = lax.axis_index("s") * 2 + lax.axis_index("c")
        pltpu.sync_copy(grows_hbm.at[pl.ds(wid * _BPW * _GPB, _BPW * _GPB)],
                        idx_v)
        pltpu.sync_copy(wrep_hbm.at[pl.ds(wid * _BPW * _KP, _BPW * _KP)], w_v)
        pltpu.sync_copy(orep_hbm.at[pl.ds(wid * _BPW * _KP, _BPW * _KP)],
                        off_v)
        cps = []
        for c in range(_BPW * _GPB // _GCH):
            cps.append(pltpu.async_copy(
                vview_hbm.at[idx_v.at[pl.ds(c * _GCH, _GCH)]],
                rows_v.at[pl.ds(c * _GCH, _GCH)], sem))
        for cp in cps:
            cp.wait()
        for j in range(_BPW):
            for vc in range(_VD // 16):
                acc = jnp.zeros((16,), jnp.float32)
                rowv = lax.iota(jnp.int32, 16) + (j * _GPB + vc * 16)
                for k in range(_KP):
                    vals = plsc.load_gather(
                        rows_v, [rowv + k * _VD, off_v[j * _KP + k]])
                    acc = acc + vals * w_v[j * _KP + k]
                acc_v[j, pl.ds(vc * 16, 16)] = acc
        pltpu.sync_copy(acc_v, out_hbm.at[pl.ds(wid * _BPW, _BPW)])

    return run(vview, grows_flat, wrep_flat, orep_flat)


def kernel(query, keys, values, W, U, b, w):
    e2d, w2d, grows, wrep, orep = _tc_stage(query, keys, W, U, b, w)
    attn = _sc_gather(values,
                      grows.reshape(_B * _GPB),
                      wrep.reshape(_B * _KP, 16),
                      orep.reshape(_B * _KP, 16))
    return (attn, w2d.reshape(_B, _L, 1), e2d.reshape(_B, _L, 1))


# speedup vs baseline: 1.0654x; 1.0052x over previous
import functools

import jax
import jax.numpy as jnp
from jax import lax
from jax.experimental import pallas as pl
from jax.experimental.pallas import tpu as pltpu
from jax.experimental.pallas import tpu_sc as plsc

_B = 128
_L = 8192
_QS = 64
_KS = 64
_BN = 64
_VD = 64
_TOPK = 20
_KP = 24
_BB = 8
_TB = 128
_NW = 32
_BPW = _B // _NW
_GPB = _KP * _VD
_GCH = 128


def _energy_body(qt_ref, kt_ref, Wt_ref, Ut_ref, bc_ref, wc_ref,
                 eo_ref, wt_ref):
    wht = jnp.dot(Wt_ref[...].astype(jnp.bfloat16),
                  qt_ref[...].astype(jnp.bfloat16),
                  preferred_element_type=jnp.float32)
    ut = Ut_ref[...].astype(jnp.bfloat16)
    bc = bc_ref[...]
    wc = wc_ref[...].astype(jnp.bfloat16).astype(jnp.float32)
    col = lax.broadcasted_iota(jnp.int32, (_BN, _B), 1)
    rows = []
    for i in range(_BB):
        gi = pl.program_id(0) * _BB + i
        whi = jnp.sum(jnp.where(col == gi, wht, 0.0), axis=1, keepdims=True)
        a = kt_ref[i].astype(jnp.bfloat16)
        uvt = jnp.dot(ut, a, preferred_element_type=jnp.float32)
        th = jnp.tanh(whi + uvt + bc)
        tb = th.astype(jnp.bfloat16).astype(jnp.float32)
        rows.append(jnp.sum(tb * wc, axis=0, keepdims=True))
    e = jnp.concatenate(rows, axis=0)
    m = jnp.max(e, axis=1, keepdims=True)
    ex = jnp.exp(e - m)
    s = jnp.sum(ex, axis=1, keepdims=True)
    eo_ref[...] = e.reshape(_BB * 64, 128)
    wt_ref[...] = (ex / s).reshape(_BB * 64, 128)


def _idx_body(eo_ref, gr_ref, wr_ref, or_ref):
    e = eo_ref[...].reshape(_TB, _L)
    m = jnp.max(e, axis=1, keepdims=True)
    ex = jnp.exp(e - m)
    s = jnp.sum(ex, axis=1, keepdims=True)

    iota = lax.broadcasted_iota(jnp.int32, (_TB, _L), 1)
    k_iota = lax.broadcasted_iota(jnp.int32, (_TB, _KP), 1)
    ecur = e
    topv = jnp.full((_TB, _KP), -jnp.inf, jnp.float32)
    topi = jnp.zeros((_TB, _KP), jnp.int32)
    for k in range(_TOPK):
        mv = jnp.max(ecur, axis=1, keepdims=True)
        cand = jnp.where(ecur == mv, iota, jnp.int32(_L))
        mi = jnp.min(cand, axis=1, keepdims=True)
        topv = jnp.where(k_iota == k, mv, topv)
        topi = jnp.where(k_iota == k, mi, topi)
        ecur = jnp.where(iota == mi, -jnp.inf, ecur)

    topw = jnp.where(topv == -jnp.inf, 0.0, jnp.exp(topv - m) / s)
    wr_ref[...] = jnp.broadcast_to(topw[:, :, None], (_TB, _KP, 16))

    b3 = (pl.program_id(0) * _TB
          + lax.broadcasted_iota(jnp.int32, (_TB, _KP, _VD), 0))
    v3 = lax.broadcasted_iota(jnp.int32, (_TB, _KP, _VD), 2)
    l3 = topi[:, :, None]
    gr_ref[...] = ((((b3 * 8 + (v3 >> 3)) * 64 + (l3 >> 7)) * 8
                    + (v3 & 7)) * 8 + ((l3 >> 4) & 7))
    or_ref[...] = jnp.broadcast_to((topi & 15)[:, :, None], (_TB, _KP, 16))


def _tc_stage(query, keys, W, U, b, w, interpret=False):
    keys_t = jnp.transpose(keys, (0, 2, 1))
    query_t = query.T
    e2d, w2d = pl.pallas_call(
        _energy_body,
        grid=(_B // _BB,),
        in_specs=[
            pl.BlockSpec((_QS, _B), lambda i: (0, 0)),
            pl.BlockSpec((_BB, _KS, _L), lambda i: (i, 0, 0)),
            pl.BlockSpec((_BN, _QS), lambda i: (0, 0)),
            pl.BlockSpec((_BN, _KS), lambda i: (0, 0)),
            pl.BlockSpec((_BN, 1), lambda i: (0, 0)),
            pl.BlockSpec((_BN, 1), lambda i: (0, 0)),
        ],
        out_specs=[
            pl.BlockSpec((_BB * 64, 128), lambda i: (i, 0)),
            pl.BlockSpec((_BB * 64, 128), lambda i: (i, 0)),
        ],
        out_shape=[
            jax.ShapeDtypeStruct((_B * 64, 128), jnp.float32),
            jax.ShapeDtypeStruct((_B * 64, 128), jnp.float32),
        ],
        compiler_params=pltpu.CompilerParams(
            vmem_limit_bytes=56 * 1024 * 1024),
        interpret=interpret,
    )(query_t, keys_t, W.T, U.T, b.reshape(_BN, 1), w.reshape(_BN, 1))

    grows, wrep, orep = pl.pallas_call(
        _idx_body,
        grid=(_B // _TB,),
        in_specs=[pl.BlockSpec((_TB * 64, 128), lambda i: (i, 0))],
        out_specs=[
            pl.BlockSpec((_TB, _KP, _VD), lambda i: (i, 0, 0)),
            pl.BlockSpec((_TB, _KP, 16), lambda i: (i, 0, 0)),
            pl.BlockSpec((_TB, _KP, 16), lambda i: (i, 0, 0)),
        ],
        out_shape=[
            jax.ShapeDtypeStruct((_B, _KP, _VD), jnp.int32),
            jax.ShapeDtypeStruct((_B, _KP, 16), jnp.float32),
            jax.ShapeDtypeStruct((_B, _KP, 16), jnp.int32),
        ],
        interpret=interpret,
    )(e2d)
    return e2d, w2d, grows, wrep, orep


def _sc_gather(values, grows_flat, wrep_flat, orep_flat):
    vview = (values.transpose(0, 2, 1)
             .reshape(_B, 8, 8, 64, 128)
             .transpose(0, 1, 3, 2, 4)
             .reshape(_B * 8 * 64 * 8 * 8, 16))
    mesh = plsc.VectorSubcoreMesh(core_axis_name="c", subcore_axis_name="s")

    @functools.partial(
        pl.kernel,
        mesh=mesh,
        out_type=jax.ShapeDtypeStruct((_B, _VD), jnp.float32),
        compiler_params=pltpu.CompilerParams(use_tc_tiling_on_sc=False,
                                             needs_layout_passes=False),
        scratch_types=[
            pltpu.VMEM((_BPW * _GPB,), jnp.int32),
            pltpu.VMEM((_BPW * _GPB, 16), jnp.float32),
            pltpu.VMEM((_BPW * _KP, 16), jnp.float32),
            pltpu.VMEM((_BPW * _KP, 16), jnp.int32),
            pltpu.VMEM((_BPW, _VD), jnp.float32),
            pltpu.SemaphoreType.DMA,
        ],
    )
    def run(vview_hbm, grows_hbm, wrep_hbm, orep_hbm, out_hbm,
            idx_v, rows_v, w_v, off_v, acc_v, sem):
        wid = lax.axis_index("s") * 2 + lax.axis_index("c")
        pltpu.sync_copy(grows_hbm.at[pl.ds(wid * _BPW * _GPB, _BPW * _GPB)],
                        idx_v)
        pltpu.sync_copy(wrep_hbm.at[pl.ds(wid * _BPW * _KP, _BPW * _KP)], w_v)
        pltpu.sync_copy(orep_hbm.at[pl.ds(wid * _BPW * _KP, _BPW * _KP)],
                        off_v)
        cps = []
        for c in range(_BPW * _GPB // _GCH):
            cps.append(pltpu.async_copy(
                vview_hbm.at[idx_v.at[pl.ds(c * _GCH, _GCH)]],
                rows_v.at[pl.ds(c * _GCH, _GCH)], sem))
        for cp in cps:
            cp.wait()
        for j in range(_BPW):
            for vc in range(_VD // 16):
                acc = jnp.zeros((16,), jnp.float32)
                rowv = lax.iota(jnp.int32, 16) + (j * _GPB + vc * 16)
                for k in range(_KP):
                    vals = plsc.load_gather(
                        rows_v, [rowv + k * _VD, off_v[j * _KP + k]])
                    acc = acc + vals * w_v[j * _KP + k]
                acc_v[j, pl.ds(vc * 16, 16)] = acc
        pltpu.sync_copy(acc_v, out_hbm.at[pl.ds(wid * _BPW, _BPW)])

    return run(vview, grows_flat, wrep_flat, orep_flat)


def kernel(query, keys, values, W, U, b, w):
    e2d, w2d, grows, wrep, orep = _tc_stage(query, keys, W, U, b, w)
    attn = _sc_gather(values,
                      grows.reshape(_B * _GPB),
                      wrep.reshape(_B * _KP, 16),
                      orep.reshape(_B * _KP, 16))
    return (attn, w2d.reshape(_B, _L, 1), e2d.reshape(_B, _L, 1))
